# baseline (speedup 1.0000x reference)
import functools

import jax
import jax.numpy as jnp
from jax.experimental import pallas as pl
from jax.experimental.pallas import tpu as pltpu

N_TOKENS = 8192
INPUT_SIZE = 4096
HIDDEN_SIZE = 2048
NUM_EXPERTS = 64

M_TILE = 2048
K_TILE = 512
K_STEPS = INPUT_SIZE // K_TILE


def _gating_kernel(x_ref, w1_ref, b1_ref, w2_ref, b2_ref, out_ref, acc_ref):
    k = pl.program_id(1)

    part = jnp.dot(x_ref[...], w1_ref[...], preferred_element_type=jnp.float32)

    @pl.when(k == 0)
    def _init():
        acc_ref[...] = part

    @pl.when(k != 0)
    def _acc():
        acc_ref[...] += part

    @pl.when(k == K_STEPS - 1)
    def _finish():
        h = jnp.maximum(acc_ref[...] + b1_ref[...], 0.0)
        logits = jnp.dot(h, w2_ref[...], preferred_element_type=jnp.float32)
        logits = logits + b2_ref[...]
        sel = jnp.argmax(logits, axis=1)
        cols = jax.lax.broadcasted_iota(jnp.int32, (M_TILE, NUM_EXPERTS), 1)
        out_ref[...] = (cols == sel[:, None]).astype(jnp.float32)


@functools.partial(jax.jit, static_argnames=())
def kernel(features, W1, b1, W2, b2):
    b1r = b1.reshape(1, HIDDEN_SIZE)
    b2r = b2.reshape(1, NUM_EXPERTS)
    grid = (N_TOKENS // M_TILE, K_STEPS)
    return pl.pallas_call(
        _gating_kernel,
        grid=grid,
        in_specs=[
            pl.BlockSpec((M_TILE, K_TILE), lambda m, k: (m, k)),
            pl.BlockSpec((K_TILE, HIDDEN_SIZE), lambda m, k: (k, 0)),
            pl.BlockSpec((1, HIDDEN_SIZE), lambda m, k: (0, 0)),
            pl.BlockSpec((HIDDEN_SIZE, NUM_EXPERTS), lambda m, k: (0, 0)),
            pl.BlockSpec((1, NUM_EXPERTS), lambda m, k: (0, 0)),
        ],
        out_specs=pl.BlockSpec((M_TILE, NUM_EXPERTS), lambda m, k: (m, 0)),
        out_shape=jax.ShapeDtypeStruct((N_TOKENS, NUM_EXPERTS), jnp.float32),
        scratch_shapes=[pltpu.VMEM((M_TILE, HIDDEN_SIZE), jnp.float32)],
        compiler_params=pltpu.CompilerParams(
            dimension_semantics=("parallel", "arbitrary"),
        ),
    )(features, W1, b1r, W2, b2r)


# speedup vs baseline: 1.1250x; 1.1250x over previous
import functools

import jax
import jax.numpy as jnp
from jax.experimental import pallas as pl
from jax.experimental.pallas import tpu as pltpu

N_TOKENS = 8192
INPUT_SIZE = 4096
HIDDEN_SIZE = 2048
NUM_EXPERTS = 64

M_TILE = 1024
K_TILE = 1024
K_STEPS = INPUT_SIZE // K_TILE


def _gating_kernel(x_ref, w1_ref, b1_ref, w2_ref, b2_ref, out_ref, acc_ref):
    k = pl.program_id(1)

    part = jnp.dot(x_ref[...], w1_ref[...], preferred_element_type=jnp.float32)

    @pl.when(k == 0)
    def _init():
        acc_ref[...] = part

    @pl.when(k != 0)
    def _acc():
        acc_ref[...] += part

    @pl.when(k == K_STEPS - 1)
    def _finish():
        h = jnp.maximum(acc_ref[...] + b1_ref[...], 0.0)
        logits = jnp.dot(h, w2_ref[...], preferred_element_type=jnp.float32)
        logits = logits + b2_ref[...]
        sel = jnp.argmax(logits, axis=1)
        cols = jax.lax.broadcasted_iota(jnp.int32, (M_TILE, NUM_EXPERTS), 1)
        out_ref[...] = (cols == sel[:, None]).astype(jnp.float32)


@functools.partial(jax.jit, static_argnames=())
def kernel(features, W1, b1, W2, b2):
    b1r = b1.reshape(1, HIDDEN_SIZE)
    b2r = b2.reshape(1, NUM_EXPERTS)
    grid = (N_TOKENS // M_TILE, K_STEPS)
    return pl.pallas_call(
        _gating_kernel,
        grid=grid,
        in_specs=[
            pl.BlockSpec((M_TILE, K_TILE), lambda m, k: (m, k)),
            pl.BlockSpec((K_TILE, HIDDEN_SIZE), lambda m, k: (k, 0)),
            pl.BlockSpec((1, HIDDEN_SIZE), lambda m, k: (0, 0)),
            pl.BlockSpec((HIDDEN_SIZE, NUM_EXPERTS), lambda m, k: (0, 0)),
            pl.BlockSpec((1, NUM_EXPERTS), lambda m, k: (0, 0)),
        ],
        out_specs=pl.BlockSpec((M_TILE, NUM_EXPERTS), lambda m, k: (m, 0)),
        out_shape=jax.ShapeDtypeStruct((N_TOKENS, NUM_EXPERTS), jnp.float32),
        scratch_shapes=[pltpu.VMEM((M_TILE, HIDDEN_SIZE), jnp.float32)],
        compiler_params=pltpu.CompilerParams(
            dimension_semantics=("parallel", "arbitrary"),
        ),
    )(features, W1, b1r, W2, b2r)


# speedup vs baseline: 1.2479x; 1.1093x over previous
import functools

import jax
import jax.numpy as jnp
from jax.experimental import pallas as pl
from jax.experimental.pallas import tpu as pltpu

N_TOKENS = 8192
INPUT_SIZE = 4096
HIDDEN_SIZE = 2048
NUM_EXPERTS = 64

M_TILE = 512


def _gating_kernel(x_ref, w1_ref, b1_ref, w2_ref, b2_ref, out_ref):
    pre = jnp.dot(x_ref[...], w1_ref[...], preferred_element_type=jnp.float32)
    h = jnp.maximum(pre + b1_ref[...], 0.0)
    logits = jnp.dot(h, w2_ref[...], preferred_element_type=jnp.float32)
    logits = logits + b2_ref[...]
    sel = jnp.argmax(logits, axis=1)
    cols = jax.lax.broadcasted_iota(jnp.int32, (M_TILE, NUM_EXPERTS), 1)
    out_ref[...] = (cols == sel[:, None]).astype(jnp.float32)


@functools.partial(jax.jit, static_argnames=())
def kernel(features, W1, b1, W2, b2):
    b1r = b1.reshape(1, HIDDEN_SIZE)
    b2r = b2.reshape(1, NUM_EXPERTS)
    grid = (N_TOKENS // M_TILE,)
    return pl.pallas_call(
        _gating_kernel,
        grid=grid,
        in_specs=[
            pl.BlockSpec((M_TILE, INPUT_SIZE), lambda m: (m, 0)),
            pl.BlockSpec((INPUT_SIZE, HIDDEN_SIZE), lambda m: (0, 0)),
            pl.BlockSpec((1, HIDDEN_SIZE), lambda m: (0, 0)),
            pl.BlockSpec((HIDDEN_SIZE, NUM_EXPERTS), lambda m: (0, 0)),
            pl.BlockSpec((1, NUM_EXPERTS), lambda m: (0, 0)),
        ],
        out_specs=pl.BlockSpec((M_TILE, NUM_EXPERTS), lambda m: (m, 0)),
        out_shape=jax.ShapeDtypeStruct((N_TOKENS, NUM_EXPERTS), jnp.float32),
        compiler_params=pltpu.CompilerParams(
            dimension_semantics=("arbitrary",),
        ),
    )(features, W1, b1r, W2, b2r)
